# Initial kernel scaffold; baseline (speedup 1.0000x reference)
import jax
import jax.numpy as jnp
from jax.experimental import pallas as pl


def kernel(input_spikes, weight):
    raise NotImplementedError("write your pallas kernel here")


# speedup vs baseline: 26.9279x; 26.9279x over previous
import numpy as np
import jax
import jax.numpy as jnp
from jax.experimental import pallas as pl
from jax.experimental.pallas import tpu as pltpu

W_MAX = 8
STEP = 1
LEAK = 2
KS = (W_MAX - 1) * (STEP + LEAK)
THETA = 512
FODEP = KS
NEURONS = 512
SYNAPSES = 512
BATCH = 32
TIME = 64
TOUT = TIME + KS + 1
TP = 128
NV = W_MAX - 1
CB = 8
NCHUNK = BATCH // CB


def _base_table():
    t = np.arange(KS, dtype=np.float64)[None, :]
    w = np.arange(W_MAX, dtype=np.float64)[:, None]
    w_step = np.maximum(np.floor(1.0 + t / STEP), 0.0)
    w_leak = np.maximum(np.ceil(w + ((w - 1.0) * STEP - t) / LEAK), 0.0)
    return np.minimum(w_step, w_leak).astype(np.int64)


def _a_cat():
    base = _base_table()
    A = np.zeros((TP, NV * TIME), dtype=np.float32)
    for s in range(NV):
        v = s + 1
        for tp in range(TOUT):
            lo = max(0, tp - 1 - (KS - 1))
            hi = min(TIME - 1, tp - 1)
            for t in range(lo, hi + 1):
                A[tp, s * TIME + t] = float(base[v, tp - 1 - t])
    return A


def _fc_kernel(xt_ref, wt_ref, acat_ref, out_ref,
               masks_ref, u_ref, elig_ref, fires_ref):
    wt = wt_ref[...]
    for s in range(NV):
        masks_ref[s] = (wt == (s + 1)).astype(jnp.bfloat16)

    iota_n = jax.lax.broadcasted_iota(jnp.int32, (TP, NEURONS), 1)
    maxks = []
    for c in range(NCHUNK):
        xc = xt_ref[pl.ds(c * CB * TIME, CB * TIME), :]
        for s in range(NV):
            u = jnp.dot(xc, masks_ref[s],
                        preferred_element_type=jnp.float32)
            u_ref[:, pl.ds(s * TIME, TIME), :] = u.reshape(CB, TIME, NEURONS)
        for bl in range(CB):
            ub = u_ref[bl]
            pot = jnp.dot(acat_ref[...], ub,
                          preferred_element_type=jnp.float32)
            pot_i = pot.astype(jnp.int32)
            key = pot_i * 1024 + (1023 - iota_n)
            maxks.append(jnp.max(key, axis=1, keepdims=True))

    keys = jnp.concatenate(maxks, axis=1)
    win = 1023 - (keys & 1023)
    elig_ref[...] = (keys >> 10 > THETA).astype(jnp.int32)

    def body(t, dep):
        e = elig_ref[pl.ds(t, 1), :]
        fire = jnp.where((e > 0) & (dep == 0), 1, 0)
        fires_ref[pl.ds(t, 1), :] = fire
        return jnp.maximum(dep + fire * (FODEP + 1) - 1, 0)

    jax.lax.fori_loop(0, TOUT, body, jnp.zeros((1, BATCH), jnp.int32))

    fires = fires_ref[...]
    iota_n3 = jax.lax.broadcasted_iota(jnp.int32, (TP, BATCH, NEURONS), 2)
    hit = (win[:, :, None] == iota_n3) & (fires[:, :, None] > 0)
    out_ref[...] = hit.astype(jnp.int32)


def kernel(input_spikes, weight):
    B, C, S, T = input_spikes.shape
    x = input_spikes.reshape(B, C * S, T)
    xt = x.transpose(0, 2, 1).reshape(B * T, C * S).astype(jnp.bfloat16)
    wtT = weight.T.astype(jnp.int32)
    acat = jnp.asarray(_a_cat())

    out3 = pl.pallas_call(
        _fc_kernel,
        out_shape=jax.ShapeDtypeStruct((TP, BATCH, NEURONS), jnp.int32),
        scratch_shapes=[
            pltpu.VMEM((NV, SYNAPSES, NEURONS), jnp.bfloat16),
            pltpu.VMEM((CB, NV * TIME, NEURONS), jnp.float32),
            pltpu.VMEM((TP, BATCH), jnp.int32),
            pltpu.VMEM((TP, BATCH), jnp.int32),
        ],
    )(xt, wtT, acat)

    out = out3[:TOUT].transpose(1, 2, 0)
    return out.reshape(B, 1, NEURONS, TOUT)
